# Initial kernel scaffold; baseline (speedup 1.0000x reference)
import jax
import jax.numpy as jnp
from jax.experimental import pallas as pl


def kernel(x, adj, weight, bias):
    raise NotImplementedError("write your pallas kernel here")


# speedup vs baseline: 1.0377x; 1.0377x over previous
import jax
import jax.numpy as jnp
from jax.experimental import pallas as pl
from jax.experimental.pallas import tpu as pltpu

_BM = 400


def _gcn_kernel(x_ref, w_ref, b_ref, adj_ref, out_ref, support_ref):
    @pl.when(pl.program_id(0) == 0)
    def _():
        support = jnp.dot(
            x_ref[...].astype(jnp.bfloat16),
            w_ref[...].astype(jnp.bfloat16),
            preferred_element_type=jnp.float32,
        )
        support_ref[...] = support.astype(jnp.bfloat16)

    acc = jnp.dot(
        adj_ref[...].astype(jnp.bfloat16),
        support_ref[...],
        preferred_element_type=jnp.float32,
    )
    out_ref[...] = acc + b_ref[...]


def kernel(x, adj, weight, bias):
    n, in_f = x.shape
    out_f = weight.shape[1]
    bias2d = bias.reshape(1, out_f)
    return pl.pallas_call(
        _gcn_kernel,
        grid=(n // _BM,),
        in_specs=[
            pl.BlockSpec((n, in_f), lambda i: (0, 0)),
            pl.BlockSpec((in_f, out_f), lambda i: (0, 0)),
            pl.BlockSpec((1, out_f), lambda i: (0, 0)),
            pl.BlockSpec((_BM, n), lambda i: (i, 0)),
        ],
        out_specs=pl.BlockSpec((_BM, out_f), lambda i: (i, 0)),
        out_shape=jax.ShapeDtypeStruct((n, out_f), jnp.float32),
        scratch_shapes=[pltpu.VMEM((n, out_f), jnp.bfloat16)],
        compiler_params=pltpu.CompilerParams(dimension_semantics=("arbitrary",)),
    )(x, weight, bias2d, adj)
